# Initial kernel scaffold; baseline (speedup 1.0000x reference)
import jax
import jax.numpy as jnp
from jax.experimental import pallas as pl


def kernel(x, edge_index, W1, b1, W2, b2):
    raise NotImplementedError("write your pallas kernel here")


# speedup vs baseline: 13.6060x; 13.6060x over previous
import dataclasses
import functools

import jax
import jax.numpy as jnp
from jax import lax
from jax.experimental import pallas as pl
from jax.experimental.pallas import tpu as pltpu
from jax.experimental.pallas import tpu_sc as plsc

_N = 10000
_NP = 10240
_E = 320000
_D = 128

_NC = 2
_NS = 16
_NW = _NC * _NS
_EPW = _E // _NW
_CH = 80
_NCHUNK = _EPW // _CH
_RPS = _NP // _NS
_HR = 80

_mesh = plsc.VectorSubcoreMesh(core_axis_name="c", subcore_axis_name="s")

_sc_params = pltpu.CompilerParams()
if "needs_layout_passes" in pltpu.CompilerParams.__dataclass_fields__:
    _sc_params = dataclasses.replace(_sc_params, needs_layout_passes=False)


def _deg_body(dst_hbm, zeros_hbm, row80_hbm, degp_hbm,
              hist_v, didx_v, row80_v, shist, sem):
    c = lax.axis_index("c")
    s = lax.axis_index("s")
    wid = c * _NS + s

    pltpu.sync_copy(zeros_hbm.at[pl.ds(0, _HR)], hist_v)

    @pl.when(s == 0)
    def _():
        pltpu.sync_copy(zeros_hbm.at[pl.ds(0, _HR)], shist)

    pltpu.sync_copy(dst_hbm.at[pl.ds(wid * _EPW, _EPW)], didx_v)
    pltpu.sync_copy(row80_hbm, row80_v)

    ones = jnp.full((16,), 1.0, jnp.float32)

    @pl.loop(0, _EPW // 16)
    def _(i):
        idx = didx_v[pl.ds(i * 16, 16)]
        r = lax.shift_right_logical(idx, 7)
        col = lax.bitwise_and(idx, 127)
        plsc.addupdate_scatter(hist_v, [r, col], ones)

    plsc.subcore_barrier()
    pltpu.sync_copy(hist_v, shist.at[row80_v], add=True)
    plsc.subcore_barrier()

    @pl.when(s == 0)
    def _():
        pltpu.sync_copy(shist, degp_hbm.at[c])


@jax.jit
def _deg(dst, zeros, row80):
    k = pl.kernel(
        _deg_body,
        out_type=jax.ShapeDtypeStruct((_NC, _HR, _D), jnp.float32),
        mesh=_mesh,
        scratch_types=[
            pltpu.VMEM((_HR, _D), jnp.float32),
            pltpu.VMEM((_EPW,), jnp.int32),
            pltpu.VMEM((_HR,), jnp.int32),
            pltpu.VMEM_SHARED((_HR, _D), jnp.float32),
            pltpu.SemaphoreType.DMA,
        ],
        compiler_params=_sc_params,
    )
    return k(dst, zeros, row80)


def _agg_body(hp_hbm, src_hbm, dst_hbm, zeros_hbm, out_hbm,
              acc, sidx, didx, rows, gsem):
    c = lax.axis_index("c")
    s = lax.axis_index("s")
    wid = c * _NS + s
    rbase = s * _RPS

    @pl.when(c == 0)
    def _():
        pltpu.sync_copy(hp_hbm.at[pl.ds(rbase, _RPS)], acc.at[pl.ds(rbase, _RPS)])

    @pl.when(c == 1)
    def _():
        pltpu.sync_copy(zeros_hbm.at[pl.ds(rbase, _RPS)], acc.at[pl.ds(rbase, _RPS)])

    plsc.subcore_barrier()

    @pl.loop(0, _NCHUNK)
    def _(i):
        base = wid * _EPW + i * _CH
        pltpu.sync_copy(src_hbm.at[pl.ds(base, _CH)], sidx)
        pltpu.sync_copy(dst_hbm.at[pl.ds(base, _CH)], didx)
        pltpu.async_copy(hp_hbm.at[sidx], rows, gsem).wait()
        pltpu.sync_copy(rows, acc.at[didx], add=True)

    plsc.subcore_barrier()
    pltpu.sync_copy(acc.at[pl.ds(rbase, _RPS)],
                    out_hbm.at[c, pl.ds(rbase, _RPS)])


@jax.jit
def _agg(hp, src, dst, zeros):
    k = pl.kernel(
        _agg_body,
        out_type=jax.ShapeDtypeStruct((_NC, _NP, _D), jnp.float32),
        mesh=_mesh,
        scratch_types=[
            pltpu.VMEM_SHARED((_NP, _D), jnp.float32),
            pltpu.VMEM((_CH,), jnp.int32),
            pltpu.VMEM((_CH,), jnp.int32),
            pltpu.VMEM((_CH, _D), jnp.float32),
            pltpu.SemaphoreType.DMA,
        ],
        compiler_params=_sc_params,
    )
    return k(hp, src, dst, zeros)


_RB = 1024


def _mm_body(x_ref, w_ref, o_ref):
    o_ref[...] = jnp.dot(x_ref[...], w_ref[...],
                         preferred_element_type=jnp.float32,
                         precision=lax.Precision.HIGHEST)


@jax.jit
def _matmul(x, w):
    return pl.pallas_call(
        _mm_body,
        grid=(_NP // _RB,),
        in_specs=[
            pl.BlockSpec((_RB, _D), lambda i: (i, 0)),
            pl.BlockSpec((_D, _D), lambda i: (0, 0)),
        ],
        out_specs=pl.BlockSpec((_RB, _D), lambda i: (i, 0)),
        out_shape=jax.ShapeDtypeStruct((_NP, _D), jnp.float32),
    )(x, w)


def _scale_body(h_ref, d0_ref, d1_ref, hp_ref, dinv_ref):
    dv = lax.rsqrt(d0_ref[...] + d1_ref[...] + 1.0)
    dinv_ref[...] = dv
    hp_ref[...] = h_ref[...] * dv


@jax.jit
def _scale(h, d0, d1):
    return pl.pallas_call(
        _scale_body,
        grid=(_NP // _RB,),
        in_specs=[
            pl.BlockSpec((_RB, _D), lambda i: (i, 0)),
            pl.BlockSpec((_RB, 1), lambda i: (i, 0)),
            pl.BlockSpec((_RB, 1), lambda i: (i, 0)),
        ],
        out_specs=[
            pl.BlockSpec((_RB, _D), lambda i: (i, 0)),
            pl.BlockSpec((_RB, 1), lambda i: (i, 0)),
        ],
        out_shape=[
            jax.ShapeDtypeStruct((_NP, _D), jnp.float32),
            jax.ShapeDtypeStruct((_NP, 1), jnp.float32),
        ],
    )(h, d0, d1)


def _mid_body(p0_ref, p1_ref, dinv_ref, b1_ref, w2_ref, o_ref):
    u = jnp.maximum(dinv_ref[...] * (p0_ref[...] + p1_ref[...]) + b1_ref[...],
                    0.0)
    o_ref[...] = jnp.dot(u, w2_ref[...],
                         preferred_element_type=jnp.float32,
                         precision=lax.Precision.HIGHEST) * dinv_ref[...]


@jax.jit
def _mid(p0, p1, dinv, b1, w2):
    return pl.pallas_call(
        _mid_body,
        grid=(_NP // _RB,),
        in_specs=[
            pl.BlockSpec((_RB, _D), lambda i: (i, 0)),
            pl.BlockSpec((_RB, _D), lambda i: (i, 0)),
            pl.BlockSpec((_RB, 1), lambda i: (i, 0)),
            pl.BlockSpec((1, _D), lambda i: (0, 0)),
            pl.BlockSpec((_D, _D), lambda i: (0, 0)),
        ],
        out_specs=pl.BlockSpec((_RB, _D), lambda i: (i, 0)),
        out_shape=jax.ShapeDtypeStruct((_NP, _D), jnp.float32),
    )(p0, p1, dinv, b1, w2)


def _fin_body(q0_ref, q1_ref, dinv_ref, b2_ref, o_ref):
    o_ref[...] = dinv_ref[...] * (q0_ref[...] + q1_ref[...]) + b2_ref[...]


@jax.jit
def _fin(q0, q1, dinv, b2):
    return pl.pallas_call(
        _fin_body,
        grid=(_NP // _RB,),
        in_specs=[
            pl.BlockSpec((_RB, _D), lambda i: (i, 0)),
            pl.BlockSpec((_RB, _D), lambda i: (i, 0)),
            pl.BlockSpec((_RB, 1), lambda i: (i, 0)),
            pl.BlockSpec((1, _D), lambda i: (0, 0)),
        ],
        out_specs=pl.BlockSpec((_RB, _D), lambda i: (i, 0)),
        out_shape=jax.ShapeDtypeStruct((_NP, _D), jnp.float32),
    )(q0, q1, dinv, b2)


def kernel(x, edge_index, W1, b1, W2, b2):
    src = edge_index[0]
    dst = edge_index[1]
    xp = jnp.pad(x, ((0, _NP - _N), (0, 0)))
    zeros = jnp.zeros((_NP, _D), jnp.float32)
    row80 = jnp.arange(_HR, dtype=jnp.int32)
    b1r = b1.reshape(1, _D)
    b2r = b2.reshape(1, _D)

    degp = _deg(dst, zeros, row80)
    h1 = _matmul(xp, W1)
    d0 = degp[0].reshape(-1)[:, None]
    d1 = degp[1].reshape(-1)[:, None]
    h1p, dinv = _scale(h1, d0, d1)
    p = _agg(h1p, src, dst, zeros)
    h2p = _mid(p[0], p[1], dinv, b1r, W2)
    q = _agg(h2p, src, dst, zeros)
    out = _fin(q[0], q[1], dinv, b2r)
    return out[:_N]


# speedup vs baseline: 29.2907x; 2.1528x over previous
import dataclasses
import functools

import jax
import jax.numpy as jnp
from jax import lax
from jax.experimental import pallas as pl
from jax.experimental.pallas import tpu as pltpu
from jax.experimental.pallas import tpu_sc as plsc

_N = 10000
_NP = 10240
_E = 320000
_D = 128

_NC = 2
_NS = 16
_NW = _NC * _NS
_EPW = _E // _NW
_CH = 80
_NCHUNK = 126
_EPWP = _NCHUNK * _CH
_EP = _NW * _EPWP
_RING = 3
_ISLOT = 6
_RPS = _NP // _NS
_HR = 80

_mesh = plsc.VectorSubcoreMesh(core_axis_name="c", subcore_axis_name="s")

_sc_params = pltpu.CompilerParams()
if "needs_layout_passes" in pltpu.CompilerParams.__dataclass_fields__:
    _sc_params = dataclasses.replace(_sc_params, needs_layout_passes=False)


def _deg_body(dst_hbm, zeros_hbm, row80_hbm, degp_hbm,
              hist_v, didx_v, row80_v, shist, sem):
    c = lax.axis_index("c")
    s = lax.axis_index("s")
    wid = c * _NS + s

    pltpu.sync_copy(zeros_hbm.at[pl.ds(0, _HR)], hist_v)

    @pl.when(s == 0)
    def _():
        pltpu.sync_copy(zeros_hbm.at[pl.ds(0, _HR)], shist)

    pltpu.sync_copy(dst_hbm.at[pl.ds(wid * _EPW, _EPW)], didx_v)
    pltpu.sync_copy(row80_hbm, row80_v)

    ones = jnp.full((16,), 1.0, jnp.float32)

    @pl.loop(0, _EPW // 16)
    def _(i):
        idx = didx_v[pl.ds(i * 16, 16)]
        r = lax.shift_right_logical(idx, 7)
        col = lax.bitwise_and(idx, 127)
        plsc.addupdate_scatter(hist_v, [r, col], ones)

    plsc.subcore_barrier()
    pltpu.sync_copy(hist_v, shist.at[row80_v], add=True)
    plsc.subcore_barrier()

    @pl.when(s == 0)
    def _():
        pltpu.sync_copy(shist, degp_hbm.at[c])


@jax.jit
def _deg(dst, zeros, row80):
    k = pl.kernel(
        _deg_body,
        out_type=jax.ShapeDtypeStruct((_NC, _HR, _D), jnp.float32),
        mesh=_mesh,
        scratch_types=[
            pltpu.VMEM((_HR, _D), jnp.float32),
            pltpu.VMEM((_EPW,), jnp.int32),
            pltpu.VMEM((_HR,), jnp.int32),
            pltpu.VMEM_SHARED((_HR, _D), jnp.float32),
            pltpu.SemaphoreType.DMA,
        ],
        compiler_params=_sc_params,
    )
    return k(dst, zeros, row80)


def _agg_body(hp_hbm, srcx_hbm, dstx_hbm, zeros_hbm, out_hbm, *scr):
    rows = scr[1:4]
    isrc = scr[4:10]
    idst = scr[10:16]
    gsem = scr[16:19]
    ssem = scr[19:22]
    pisem = scr[22:28]
    pjsem = scr[28:34]
    acc = scr[0]
    c = lax.axis_index("c")
    s = lax.axis_index("s")
    wid = c * _NS + s
    rbase = s * _RPS
    ebase = wid * _EPWP

    def i_issue(i, sl):
        off = ebase + i * _CH
        pltpu.async_copy(srcx_hbm.at[pl.ds(off, _CH)], isrc[sl], pisem[sl])
        pltpu.async_copy(dstx_hbm.at[pl.ds(off, _CH)], idst[sl], pjsem[sl])

    def i_wait(i, sl):
        off = ebase + i * _CH
        pltpu.make_async_copy(srcx_hbm.at[pl.ds(off, _CH)], isrc[sl], pisem[sl]).wait()
        pltpu.make_async_copy(dstx_hbm.at[pl.ds(off, _CH)], idst[sl], pjsem[sl]).wait()

    def g_start(b, sl):
        pltpu.async_copy(hp_hbm.at[isrc[sl]], rows[b], gsem[b])

    def g_wait(b, sl):
        pltpu.make_async_copy(hp_hbm.at[isrc[sl]], rows[b], gsem[b]).wait()

    def s_start(b, sl):
        pltpu.async_copy(rows[b], acc.at[idst[sl]], ssem[b], add=True)

    def s_wait(b, sl):
        pltpu.make_async_copy(rows[b], acc.at[idst[sl]], ssem[b]).wait()

    def phase(i, sl, first=False, stop_issue=False, stop_next=False):
        b = sl % 3
        g_wait(b, sl)
        s_start(b, sl)
        if not first:
            s_wait((b + 2) % 3, (sl + 5) % 6)
        if not stop_issue:
            i_issue(i + 4, (sl + 4) % 6)
        if not stop_next:
            i_wait(i + 2, (sl + 2) % 6)
            g_start((b + 2) % 3, (sl + 2) % 6)

    for i0 in range(4):
        i_issue(i0, i0)

    @pl.when(c == 0)
    def _():
        pltpu.sync_copy(hp_hbm.at[pl.ds(rbase, _RPS)], acc.at[pl.ds(rbase, _RPS)])

    @pl.when(c == 1)
    def _():
        pltpu.sync_copy(zeros_hbm.at[pl.ds(rbase, _RPS)], acc.at[pl.ds(rbase, _RPS)])

    for i0 in range(2):
        i_wait(i0, i0)
        g_start(i0, i0)

    plsc.subcore_barrier()

    for u in range(6):
        phase(u, u, first=(u == 0))

    @pl.loop(1, _NCHUNK // 6 - 1)
    def _(m):
        i = m * 6
        for u in range(6):
            phase(i + u, u)

    for u in range(6):
        i = _NCHUNK - 6 + u
        phase(i, u, stop_issue=(i + 4 >= _NCHUNK), stop_next=(i + 2 >= _NCHUNK))

    s_wait(2, 5)

    plsc.subcore_barrier()
    pltpu.sync_copy(acc.at[pl.ds(rbase, _RPS)],
                    out_hbm.at[c, pl.ds(rbase, _RPS)])


@jax.jit
def _agg(hp, srcx, dstx, zeros):
    k = pl.kernel(
        _agg_body,
        out_type=jax.ShapeDtypeStruct((_NC, _NP, _D), jnp.float32),
        mesh=_mesh,
        scratch_types=(
            [pltpu.VMEM_SHARED((_NP, _D), jnp.float32)]
            + [pltpu.VMEM((_CH, _D), jnp.float32) for _ in range(_RING)]
            + [pltpu.VMEM((_CH,), jnp.int32) for _ in range(2 * _ISLOT)]
            + [pltpu.SemaphoreType.DMA for _ in range(2 * _RING + 2 * _ISLOT)]
        ),
        compiler_params=_sc_params,
    )
    return k(hp, srcx, dstx, zeros)


_RB = 1024


def _mm_body(x_ref, w_ref, o_ref):
    o_ref[...] = jnp.dot(x_ref[...], w_ref[...],
                         preferred_element_type=jnp.float32,
                         precision=lax.Precision.HIGHEST)


@jax.jit
def _matmul(x, w):
    return pl.pallas_call(
        _mm_body,
        grid=(_NP // _RB,),
        in_specs=[
            pl.BlockSpec((_RB, _D), lambda i: (i, 0)),
            pl.BlockSpec((_D, _D), lambda i: (0, 0)),
        ],
        out_specs=pl.BlockSpec((_RB, _D), lambda i: (i, 0)),
        out_shape=jax.ShapeDtypeStruct((_NP, _D), jnp.float32),
    )(x, w)


def _scale_body(h_ref, d0_ref, d1_ref, hp_ref, dinv_ref):
    dv = lax.rsqrt(d0_ref[...] + d1_ref[...] + 1.0)
    dinv_ref[...] = dv
    hp_ref[...] = h_ref[...] * dv


@jax.jit
def _scale(h, d0, d1):
    return pl.pallas_call(
        _scale_body,
        grid=(_NP // _RB,),
        in_specs=[
            pl.BlockSpec((_RB, _D), lambda i: (i, 0)),
            pl.BlockSpec((_RB, 1), lambda i: (i, 0)),
            pl.BlockSpec((_RB, 1), lambda i: (i, 0)),
        ],
        out_specs=[
            pl.BlockSpec((_RB, _D), lambda i: (i, 0)),
            pl.BlockSpec((_RB, 1), lambda i: (i, 0)),
        ],
        out_shape=[
            jax.ShapeDtypeStruct((_NP, _D), jnp.float32),
            jax.ShapeDtypeStruct((_NP, 1), jnp.float32),
        ],
    )(h, d0, d1)


def _mid_body(p0_ref, p1_ref, dinv_ref, b1_ref, w2_ref, o_ref):
    u = jnp.maximum(dinv_ref[...] * (p0_ref[...] + p1_ref[...]) + b1_ref[...],
                    0.0)
    o_ref[...] = jnp.dot(u, w2_ref[...],
                         preferred_element_type=jnp.float32,
                         precision=lax.Precision.HIGHEST) * dinv_ref[...]


@jax.jit
def _mid(p0, p1, dinv, b1, w2):
    return pl.pallas_call(
        _mid_body,
        grid=(_NP // _RB,),
        in_specs=[
            pl.BlockSpec((_RB, _D), lambda i: (i, 0)),
            pl.BlockSpec((_RB, _D), lambda i: (i, 0)),
            pl.BlockSpec((_RB, 1), lambda i: (i, 0)),
            pl.BlockSpec((1, _D), lambda i: (0, 0)),
            pl.BlockSpec((_D, _D), lambda i: (0, 0)),
        ],
        out_specs=pl.BlockSpec((_RB, _D), lambda i: (i, 0)),
        out_shape=jax.ShapeDtypeStruct((_NP, _D), jnp.float32),
    )(p0, p1, dinv, b1, w2)


def _fin_body(q0_ref, q1_ref, dinv_ref, b2_ref, o_ref):
    o_ref[...] = dinv_ref[...] * (q0_ref[...] + q1_ref[...]) + b2_ref[...]


@jax.jit
def _fin(q0, q1, dinv, b2):
    return pl.pallas_call(
        _fin_body,
        grid=(_NP // _RB,),
        in_specs=[
            pl.BlockSpec((_RB, _D), lambda i: (i, 0)),
            pl.BlockSpec((_RB, _D), lambda i: (i, 0)),
            pl.BlockSpec((_RB, 1), lambda i: (i, 0)),
            pl.BlockSpec((1, _D), lambda i: (0, 0)),
        ],
        out_specs=pl.BlockSpec((_RB, _D), lambda i: (i, 0)),
        out_shape=jax.ShapeDtypeStruct((_NP, _D), jnp.float32),
    )(q0, q1, dinv, b2)


def kernel(x, edge_index, W1, b1, W2, b2):
    src = edge_index[0]
    dst = edge_index[1]
    xp = jnp.pad(x, ((0, _NP - _N), (0, 0)))
    zeros = jnp.zeros((_NP, _D), jnp.float32)
    row80 = jnp.arange(_HR, dtype=jnp.int32)
    b1r = b1.reshape(1, _D)
    b2r = b2.reshape(1, _D)
    padrow = 10200 + jnp.arange(_NW, dtype=jnp.int32)[:, None]
    padblk = jnp.broadcast_to(padrow, (_NW, _EPWP - _EPW))
    srcx = jnp.concatenate([src.reshape(_NW, _EPW), padblk], axis=1).reshape(-1)
    dstx = jnp.concatenate([dst.reshape(_NW, _EPW), padblk], axis=1).reshape(-1)

    degp = _deg(dst, zeros, row80)
    h1 = _matmul(xp, W1)
    d0 = degp[0].reshape(-1)[:, None]
    d1 = degp[1].reshape(-1)[:, None]
    h1p, dinv = _scale(h1, d0, d1)
    p = _agg(h1p, srcx, dstx, zeros)
    h2p = _mid(p[0], p[1], dinv, b1r, W2)
    q = _agg(h2p, srcx, dstx, zeros)
    out = _fin(q[0], q[1], dinv, b2r)
    return out[:_N]
